# Initial kernel scaffold; baseline (speedup 1.0000x reference)
import jax
import jax.numpy as jnp
from jax.experimental import pallas as pl


def kernel(x, seg, tok_embed, pos_embed, seg_embed, gamma, beta):
    raise NotImplementedError("write your pallas kernel here")


# speedup vs baseline: 3.2449x; 3.2449x over previous
import functools

import jax
import jax.numpy as jnp
from jax import lax
from jax.experimental import pallas as pl
from jax.experimental.pallas import tpu as pltpu
from jax.experimental.pallas import tpu_sc as plsc

_B, _S, _D = 4096, 20, 768
_VOCAB, _NSEG = 4, 2
_NROWS = _VOCAB * _NSEG * _S
_BS = _B * _S
_NC, _NS, _L = 2, 16, 16
_NW = _NC * _NS
_PER_W = _BS // _NW
_CHUNK = 64
_NCH = _PER_W // _CHUNK
_NVEC = _PER_W // _L


def _table_body(tok_ref, sege_ref, pos_ref, gamma_ref, beta_ref, out_ref):
    rows = lax.broadcasted_iota(jnp.int32, (_NROWS, 1), 0)
    v = rows // (_NSEG * _S)
    g = (rows // _S) % _NSEG
    s = rows % _S
    ohv = (v == lax.broadcasted_iota(jnp.int32, (_NROWS, _VOCAB), 1)).astype(jnp.float32)
    ohg = (g == lax.broadcasted_iota(jnp.int32, (_NROWS, _NSEG), 1)).astype(jnp.float32)
    ohs = (s == lax.broadcasted_iota(jnp.int32, (_NROWS, _S), 1)).astype(jnp.float32)
    emb = (
        jnp.dot(ohv, tok_ref[...], preferred_element_type=jnp.float32)
        + jnp.dot(ohg, sege_ref[...], preferred_element_type=jnp.float32)
        + jnp.dot(ohs, pos_ref[...], preferred_element_type=jnp.float32)
    )
    mean = jnp.mean(emb, axis=-1, keepdims=True)
    cent = emb - mean
    var = jnp.mean(cent * cent, axis=-1, keepdims=True)
    normed = cent * lax.rsqrt(var + 1e-5)
    out_ref[...] = normed * gamma_ref[...] + beta_ref[...]


def _build_table(tok_embed, pos20, seg_embed, gamma, beta):
    return pl.pallas_call(
        _table_body,
        out_shape=jax.ShapeDtypeStruct((_NROWS, _D), jnp.float32),
    )(tok_embed, seg_embed, pos20, gamma.reshape(1, _D), beta.reshape(1, _D))


def _sc_body(table_hbm, x_hbm, seg_hbm, out_hbm,
             xb, sb, idx, buf0, buf1, gsem0, gsem1, ssem0, ssem1):
    wid = lax.axis_index("s") * _NC + lax.axis_index("c")
    base = wid * _PER_W
    pltpu.sync_copy(x_hbm.at[pl.ds(base, _PER_W)], xb)
    pltpu.sync_copy(seg_hbm.at[pl.ds(base, _PER_W)], sb)

    lane = lax.iota(jnp.int32, _L)
    for i in range(_NVEC):
        t = base + (i * _L) + lane
        s_of_t = lax.rem(t, _S)
        cid = xb[pl.ds(i * _L, _L)] * (_NSEG * _S) + sb[pl.ds(i * _L, _L)] * _S + s_of_t
        idx[i // (_CHUNK // _L), pl.ds((i % (_CHUNK // _L)) * _L, _L)] = cid

    bufs = (buf0, buf1)
    gsems = (gsem0, gsem1)
    ssems = (ssem0, ssem1)
    gather_h = [None] * _NCH
    store_h = [None] * _NCH

    gather_h[0] = pltpu.async_copy(table_hbm.at[idx.at[0]], bufs[0], gsems[0])
    for g in range(_NCH):
        nxt = g + 1
        if nxt < _NCH:
            if nxt >= 2:
                store_h[nxt - 2].wait()
            gather_h[nxt] = pltpu.async_copy(
                table_hbm.at[idx.at[nxt]], bufs[nxt % 2], gsems[nxt % 2])
        gather_h[g].wait()
        store_h[g] = pltpu.async_copy(
            bufs[g % 2], out_hbm.at[pl.ds(base + g * _CHUNK, _CHUNK)], ssems[g % 2])
    store_h[_NCH - 2].wait()
    store_h[_NCH - 1].wait()


def _gather_rows(table, x_flat, seg_flat):
    mesh = plsc.VectorSubcoreMesh(
        core_axis_name="c", subcore_axis_name="s",
        num_cores=_NC, num_subcores=_NS)
    fn = functools.partial(
        pl.kernel,
        out_type=jax.ShapeDtypeStruct((_BS, _D), jnp.float32),
        mesh=mesh,
        scratch_types=[
            pltpu.VMEM((_PER_W,), jnp.int32),
            pltpu.VMEM((_PER_W,), jnp.int32),
            pltpu.VMEM((_NCH, _CHUNK), jnp.int32),
            pltpu.VMEM((_CHUNK, _D), jnp.float32),
            pltpu.VMEM((_CHUNK, _D), jnp.float32),
            pltpu.SemaphoreType.DMA,
            pltpu.SemaphoreType.DMA,
            pltpu.SemaphoreType.DMA,
            pltpu.SemaphoreType.DMA,
        ],
    )(_sc_body)
    return fn(table, x_flat, seg_flat)


def kernel(x, seg, tok_embed, pos_embed, seg_embed, gamma, beta):
    table = _build_table(tok_embed, pos_embed[:_S], seg_embed, gamma, beta)
    x_flat = x.reshape(_BS).astype(jnp.int32)
    seg_flat = seg.reshape(_BS).astype(jnp.int32)
    out = _gather_rows(table, x_flat, seg_flat)
    return out.reshape(_B, _S, _D)
